# baseline (speedup 1.0000x reference)
import jax
import jax.numpy as jnp
from jax.experimental import pallas as pl
from jax.experimental.pallas import tpu as pltpu

CH = 512
NBUF = 3


def _body(x_hbm, pe_hbm, o_hbm, xb, peb, ob, rx, rp, ws):
    n_chunks = x_hbm.shape[0] // CH

    def x_copy(i, slot):
        return pltpu.make_async_copy(
            x_hbm.at[pl.ds(i * CH, CH)], xb.at[slot], rx.at[slot])

    def pe_copy(i, slot):
        return pltpu.make_async_copy(
            pe_hbm.at[pl.ds(i * CH, CH)], peb.at[slot], rp.at[slot])

    def o_copy(i, slot):
        return pltpu.make_async_copy(
            ob.at[slot], o_hbm.at[pl.ds(i * CH, CH)], ws.at[slot])

    for i in range(NBUF - 1):
        x_copy(i, i).start()
        pe_copy(i, i).start()

    def step(i, carry):
        slot = jax.lax.rem(i, NBUF)
        nxt = i + NBUF - 1
        nslot = jax.lax.rem(nxt, NBUF)

        @pl.when(nxt < n_chunks)
        def _():
            x_copy(nxt, nslot).start()
            pe_copy(nxt, nslot).start()

        x_copy(i, slot).wait()
        pe_copy(i, slot).wait()

        @pl.when(i >= NBUF)
        def _():
            o_copy(i - NBUF, slot).wait()

        pe = peb.at[slot][...]
        ob.at[slot][...] = xb.at[slot][...] + pe[:, None, :]
        o_copy(i, slot).start()
        return carry

    jax.lax.fori_loop(0, n_chunks, step, 0)

    for k in range(NBUF):
        i = n_chunks - NBUF + k
        o_copy(i, i % NBUF).wait()


def kernel(x, pos_emb):
    seq_len, batch, d_model = x.shape
    return pl.pallas_call(
        _body,
        in_specs=[
            pl.BlockSpec(memory_space=pl.ANY),
            pl.BlockSpec(memory_space=pl.ANY),
        ],
        out_specs=pl.BlockSpec(memory_space=pl.ANY),
        out_shape=jax.ShapeDtypeStruct((seq_len, batch, d_model), x.dtype),
        scratch_shapes=[
            pltpu.VMEM((NBUF, CH, batch, d_model), x.dtype),
            pltpu.VMEM((NBUF, CH, d_model), x.dtype),
            pltpu.VMEM((NBUF, CH, batch, d_model), x.dtype),
            pltpu.SemaphoreType.DMA((NBUF,)),
            pltpu.SemaphoreType.DMA((NBUF,)),
            pltpu.SemaphoreType.DMA((NBUF,)),
        ],
    )(x, pos_emb)


# speedup vs baseline: 1.0093x; 1.0093x over previous
import jax
import jax.numpy as jnp
from jax.experimental import pallas as pl
from jax.experimental.pallas import tpu as pltpu

CH = 128
NBUF = 8


def _body(x_hbm, pe_hbm, o_hbm, xb, peb, ob, rx, rp, ws):
    n_chunks = x_hbm.shape[0] // CH

    def x_copy(i, slot):
        return pltpu.make_async_copy(
            x_hbm.at[pl.ds(i * CH, CH)], xb.at[slot], rx.at[slot])

    def pe_copy(i, slot):
        return pltpu.make_async_copy(
            pe_hbm.at[pl.ds(i * CH, CH)], peb.at[slot], rp.at[slot])

    def o_copy(i, slot):
        return pltpu.make_async_copy(
            ob.at[slot], o_hbm.at[pl.ds(i * CH, CH)], ws.at[slot])

    for i in range(NBUF - 1):
        x_copy(i, i).start()
        pe_copy(i, i).start()

    def step(i, carry):
        slot = jax.lax.rem(i, NBUF)
        nxt = i + NBUF - 1
        nslot = jax.lax.rem(nxt, NBUF)

        @pl.when(nxt < n_chunks)
        def _():
            x_copy(nxt, nslot).start()
            pe_copy(nxt, nslot).start()

        x_copy(i, slot).wait()
        pe_copy(i, slot).wait()

        @pl.when(i >= NBUF)
        def _():
            o_copy(i - NBUF, slot).wait()

        pe = peb.at[slot][...]
        ob.at[slot][...] = xb.at[slot][...] + pe[:, None, :]
        o_copy(i, slot).start()
        return carry

    jax.lax.fori_loop(0, n_chunks, step, 0)

    for k in range(NBUF):
        i = n_chunks - NBUF + k
        o_copy(i, i % NBUF).wait()


def kernel(x, pos_emb):
    seq_len, batch, d_model = x.shape
    return pl.pallas_call(
        _body,
        in_specs=[
            pl.BlockSpec(memory_space=pl.ANY),
            pl.BlockSpec(memory_space=pl.ANY),
        ],
        out_specs=pl.BlockSpec(memory_space=pl.ANY),
        out_shape=jax.ShapeDtypeStruct((seq_len, batch, d_model), x.dtype),
        scratch_shapes=[
            pltpu.VMEM((NBUF, CH, batch, d_model), x.dtype),
            pltpu.VMEM((NBUF, CH, d_model), x.dtype),
            pltpu.VMEM((NBUF, CH, batch, d_model), x.dtype),
            pltpu.SemaphoreType.DMA((NBUF,)),
            pltpu.SemaphoreType.DMA((NBUF,)),
            pltpu.SemaphoreType.DMA((NBUF,)),
        ],
    )(x, pos_emb)
